# baseline (speedup 1.0000x reference)
import functools

import numpy as np
import jax
import jax.numpy as jnp
from jax import lax
from jax.experimental import pallas as pl
from jax.experimental.pallas import tpu as pltpu
from jax.experimental.pallas import tpu_sc as plsc

MAX_NODES = 9
RNN_H = 32
F = 4


def _lk(x):
    return jnp.maximum(x, 0.01 * x)


_S1_MATS = [(8, 512, 256), (520, 256, 128), (776, 128, 4), (904, 512, 256),
            (1416, 256, 128), (1672, 128, 128), (1800, 256, 384),
            (2056, 128, 384), (2184, 128, 128)]
_S1_ROWS = 2312


def _stage1_body(lat_ref, wn1, bn1r, wn2, bn2r, wn3, bn3r, wltr, wi1r, bi1r,
                 wi2r, bi2r, whwr, bihr, bhhr, wlr, blr,
                 nn_ref, st_ref, scr):
    @pl.when(pl.program_id(0) == 0)
    def _prep():
        scr[...] = jnp.zeros(scr.shape, scr.dtype)
        for u in range(F):
            scr[0:1, 64 * u:64 * u + 64] = bn1r[...]
            scr[1:2, 32 * u:32 * u + 32] = bn2r[...]
            scr[2:3, u:u + 1] = bn3r[...]
            scr[3:4, 32 * u:32 * u + 32] = bi1r[...]
            scr[4:5, 32 * u:32 * u + 32] = bi2r[...]
            scr[7:8, 32 * u:32 * u + 32] = blr[...]
            for k in range(3):
                scr[5:6, 128 * k + 32 * u:128 * k + 32 * u + 32] = bihr[0:1, 32 * k:32 * k + 32]
                scr[6:7, 128 * k + 32 * u:128 * k + 32 * u + 32] = bhhr[0:1, 32 * k:32 * k + 32]
        for (off, _, _), wref, do in zip(_S1_MATS[:6] + [_S1_MATS[8]],
                                         [wn1, wn2, wn3, wltr, wi1r, wi2r, wlr],
                                         [64, 32, 1, 64, 32, 32, 32]):
            wT = wref[...].T
            di = wT.shape[0]
            for u in range(F):
                scr[pl.ds(off + di * u, di), do * u:do * u + do] = wT
        for off, wv in ((1800, whwr[:, 0:64]), (2056, whwr[:, 64:96])):
            wT = wv.T
            di = wT.shape[0]
            for u in range(F):
                for k in range(3):
                    scr[pl.ds(off + di * u, di),
                        128 * k + 32 * u:128 * k + 32 * u + 32] = wT[:, 32 * k:32 * k + 32]

    def mat(i):
        off, r, c = _S1_MATS[i]
        return scr[pl.ds(off, r), 0:c]

    def bias(i, c):
        return scr[pl.ds(i, 1), 0:c]

    a1, a2, a3 = mat(0), mat(1), mat(2)
    wlt, wi1, wi2, wih, whh, wl = mat(3), mat(4), mat(5), mat(6), mat(7), mat(8)
    c1, c2, c3 = bias(0, 256), bias(1, 128), bias(2, 4)
    bi1, bi2 = bias(3, 128), bias(4, 128)
    bih, bhh, bl = bias(5, 384), bias(6, 384), bias(7, 128)

    lat = lat_ref[...].reshape(lat_ref.shape[0] // F, F * lat_ref.shape[1])
    h = _lk(lat @ a1 + c1)
    h = _lk(h @ a2 + c2)
    nn_ref[...] = h @ a3 + c3

    proj = jax.nn.sigmoid(lat @ wlt)
    hs = jnp.maximum(proj @ wi1 + bi1, 0.0)
    hs = hs @ wi2 + bi2
    gi = proj @ wih + bih
    W = F * RNN_H
    ir = gi[:, 0:W]
    iz = gi[:, W:2 * W]
    inn = gi[:, 2 * W:3 * W]
    for t in range(MAX_NODES):
        gh = hs @ whh + bhh
        hr = gh[:, 0:W]
        hz = gh[:, W:2 * W]
        hn = gh[:, 2 * W:3 * W]
        r = jax.nn.sigmoid(ir + hr)
        z = jax.nn.sigmoid(iz + hz)
        n = jnp.tanh(inn + r * hn)
        hs = (1.0 - z) * n + z * hs
        st_ref[t] = hs @ wl + bl


def _run_stage1(latent, nn_mlp, Wlt, gru):
    Bn, L = latent.shape
    R = Bn // F
    BLK = 1024
    grid = (R // BLK,)
    Wn1, bn1, Wn2, bn2, Wn3, bn3 = nn_mlp
    Wi1, bi1, Wi2, bi2, Wih, Whh, bih, bhh, Wl, bl = gru
    raw = [Wn1, bn1[None, :], Wn2, bn2[None, :], Wn3, bn3[None, :], Wlt,
           Wi1, bi1[None, :], Wi2, bi2[None, :],
           jnp.concatenate([Wih, Whh], axis=1),
           bih[None, :], bhh[None, :], Wl, bl[None, :]]

    def full(a):
        return pl.BlockSpec(a.shape, lambda i: (0,) * a.ndim)

    nn4, states = pl.pallas_call(
        _stage1_body,
        grid=grid,
        in_specs=[pl.BlockSpec((F * BLK, L), lambda i: (i, 0))] + [full(a) for a in raw],
        out_specs=[pl.BlockSpec((BLK, F), lambda i: (i, 0)),
                   pl.BlockSpec((MAX_NODES, BLK, F * RNN_H), lambda i: (0, i, 0))],
        out_shape=[jax.ShapeDtypeStruct((R, F), jnp.float32),
                   jax.ShapeDtypeStruct((MAX_NODES, R, F * RNN_H), jnp.float32)],
        scratch_shapes=[pltpu.VMEM((_S1_ROWS, 384), jnp.float32)],
    )(latent, *raw)
    return nn4.reshape(-1), states


def _run_sc_window_sum(flat128, B, per):
    NW = 32
    gpw = B // NW
    grp = gpw // 4
    rows = grp * MAX_NODES
    mesh = plsc.VectorSubcoreMesh(core_axis_name="c", subcore_axis_name="s")

    NCH = 4
    CH = rows // NCH
    GC = grp // NCH

    @functools.partial(
        pl.kernel,
        out_type=jax.ShapeDtypeStruct((B, RNN_H), jnp.float32),
        mesh=mesh,
        scratch_types=[pltpu.VMEM((2, CH, 128), jnp.float32),
                       pltpu.VMEM((gpw, RNN_H), jnp.float32),
                       pltpu.SemaphoreType.DMA,
                       pltpu.SemaphoreType.DMA],
    )
    def _sc_sum(flat_hbm, out_hbm, bufs, acc_v, sem0, sem1):
        wid = lax.axis_index("s") * 2 + lax.axis_index("c")
        base = wid * rows
        sems = (sem0, sem1)
        handles = [None, None]
        handles[0] = pltpu.async_copy(flat_hbm.at[pl.ds(base, CH)], bufs.at[0], sems[0])
        for c in range(NCH):
            nb = (c + 1) % 2
            if c + 1 < NCH:
                handles[nb] = pltpu.async_copy(
                    flat_hbm.at[pl.ds(base + (c + 1) * CH, CH)], bufs.at[nb], sems[nb])
            handles[c % 2].wait()

            def body(q, carry, c=c):
                rbase = q * MAX_NODES
                for d in range(4):
                    for h in range(2):
                        p0 = 32 * MAX_NODES * d + 16 * h
                        acc = bufs[c % 2, rbase + p0 // 128, pl.ds(p0 % 128, 16)]
                        for j in range(1, per):
                            p = p0 + 32 * j
                            acc = acc + bufs[c % 2, rbase + p // 128, pl.ds(p % 128, 16)]
                        acc_v[4 * (GC * c + q) + d, pl.ds(16 * h, 16)] = acc
                return carry

            lax.fori_loop(0, GC, body, 0)
        pltpu.sync_copy(acc_v, out_hbm.at[pl.ds(wid * gpw, gpw)])

    return _sc_sum(flat128)


def _stage3_body(offs, per, *refs):
    s_ref = refs[0]
    colp = refs[1]
    m128 = refs[2]
    out_ref = refs[-2]
    ot_scr = refs[-1]
    wrefs = refs[3:-2]
    scale = float(per)

    @pl.when(pl.program_id(0) == 0)
    def _compute():
        def col(i):
            return colp[pl.ds(offs[i], offs[i + 1] - offs[i]), :]

        xT = s_ref[...].T
        k = 0
        c = 0
        for i in range(4):
            has_m = i < 3
            if i == 2:
                pair = wrefs[k][...]
                w1, w2 = pair[:, 0:32], pair[:, 32:96]
                k += 1
            else:
                w1 = wrefs[k][...]
                w2 = wrefs[k + 1][...]
                k += 2
            if i == 0:
                w1 = w1[:, 0:RNN_H]
                z = w1 @ xT
            else:
                z = w1 @ (scale * xT)
            b1, g1, bt1, b2 = col(c), col(c + 1), col(c + 2), col(c + 3)
            z = z + b1
            m = jnp.mean(z, axis=1, keepdims=True)
            zc = z - m
            v = jnp.mean(zc * zc, axis=1, keepdims=True)
            z = _lk(zc * (g1 / jnp.sqrt(v + 1e-5)) + bt1)
            z = w2 @ z + b2
            if has_m:
                mg, mb = col(c + 4), col(c + 5)
                c += 6
                m = jnp.mean(z, axis=1, keepdims=True)
                zc = z - m
                v = jnp.mean(zc * zc, axis=1, keepdims=True)
                z = _lk(zc * (mg / jnp.sqrt(v + 1e-5)) + mb)
            else:
                c += 4
            xT = z
        f1, f2, f3 = wrefs[k][...], wrefs[k + 1][...], wrefs[k + 2][...]
        o = _lk(f1 @ xT)
        o = _lk(f2 @ o)
        ot_scr[...] = f3 @ o

    qcols = out_ref.shape[1] // (128 * per)
    base = pl.program_id(0) * qcols
    for t in range(qcols):
        out_ref[:, per * 128 * t:per * 128 * (t + 1)] = (
            ot_scr[:, pl.ds(128 * (base + t), 128)] @ m128[...])


def _run_stage3(S, gin, fin, B, per):
    wmats, cols, offs = [], [], [0]
    for li, layer in enumerate(gin):
        if len(layer) == 8:
            W1, b1, g1, bt1, W2, b2, mg, mb = layer
            vecs = [b1, g1, bt1, b2, mg, mb]
        else:
            W1, b1, g1, bt1, W2, b2 = layer
            vecs = [b1, g1, bt1, b2]
        if li == 2:
            wmats += [jnp.concatenate([W1, W2], axis=1)]
        else:
            wmats += [W1, W2]
        for vct in vecs:
            cols.append(vct)
            offs.append(offs[-1] + vct.shape[0])
    Wf1, Wf2, Wf3 = fin
    wmats += [Wf1, Wf2, Wf3]
    colp = jnp.concatenate(cols)[:, None]

    odim = Wf3.shape[0]
    m128 = np.zeros((128, 128 * per), np.float32)
    m128[np.arange(128).repeat(per), np.arange(128 * per)] = 1.0
    m128 = jnp.asarray(m128)

    def full(a):
        return pl.BlockSpec(a.shape, lambda i: (0,) * a.ndim)

    NQ = 8
    out_t = pl.pallas_call(
        functools.partial(_stage3_body, tuple(offs), per),
        grid=(NQ,),
        in_specs=[full(S), full(colp), full(m128)] + [full(a) for a in wmats],
        out_specs=pl.BlockSpec((odim, B * per // NQ), lambda i: (0, i)),
        out_shape=jax.ShapeDtypeStruct((odim, B * per), jnp.float32),
        scratch_shapes=[pltpu.VMEM((odim, B), jnp.float32)],
    )(S, colp, m128, *wmats)
    return out_t.T


def _edge_index_np(B, per):
    ii, jj = np.meshgrid(np.arange(per), np.arange(per), indexing="ij")
    m = ii != jj
    offs = (np.arange(B, dtype=np.int64) * per)[:, None]
    src = (ii[m][None, :] + offs).reshape(-1)
    dst = (jj[m][None, :] + offs).reshape(-1)
    return np.stack([src, dst]).astype(np.int32)


def kernel(latent_vec, batch, nn_mlp, Wlt, gru, gin, fin):
    B, L = latent_vec.shape
    per = batch.shape[0] // B
    edge_index = jnp.asarray(_edge_index_np(B, per))

    number_nodes, states = _run_stage1(latent_vec, nn_mlp, Wlt, gru)
    flat128 = states.reshape(MAX_NODES * B // F, F * RNN_H)
    S = _run_sc_window_sum(flat128, B, per)
    out = _run_stage3(S, gin, fin, B, per)
    return out, edge_index, batch, number_nodes


# speedup vs baseline: 1.0199x; 1.0199x over previous
import functools

import numpy as np
import jax
import jax.numpy as jnp
from jax import lax
from jax.experimental import pallas as pl
from jax.experimental.pallas import tpu as pltpu
from jax.experimental.pallas import tpu_sc as plsc

MAX_NODES = 9
RNN_H = 32
F = 4


def _lk(x):
    return jnp.maximum(x, 0.01 * x)


_S1_MATS = [(8, 512, 256), (520, 256, 128), (776, 128, 4), (904, 512, 256),
            (1416, 256, 128), (1672, 128, 128), (1800, 256, 384),
            (2056, 128, 384), (2184, 128, 128)]
_S1_ROWS = 2312


def _stage1_body(lat_ref, wn1, bn1r, wn2, bn2r, wn3, bn3r, wltr, wi1r, bi1r,
                 wi2r, bi2r, whwr, bihr, bhhr, wlr, blr,
                 nn_ref, st_ref, scr):
    @pl.when(pl.program_id(0) == 0)
    def _prep():
        scr[...] = jnp.zeros(scr.shape, scr.dtype)
        for u in range(F):
            scr[0:1, 64 * u:64 * u + 64] = bn1r[...]
            scr[1:2, 32 * u:32 * u + 32] = bn2r[...]
            scr[2:3, u:u + 1] = bn3r[...]
            scr[3:4, 32 * u:32 * u + 32] = bi1r[...]
            scr[4:5, 32 * u:32 * u + 32] = bi2r[...]
            scr[7:8, 32 * u:32 * u + 32] = blr[...]
            for k in range(3):
                scr[5:6, 128 * k + 32 * u:128 * k + 32 * u + 32] = bihr[0:1, 32 * k:32 * k + 32]
                scr[6:7, 128 * k + 32 * u:128 * k + 32 * u + 32] = bhhr[0:1, 32 * k:32 * k + 32]
        for (off, _, _), wref, do in zip(_S1_MATS[:6] + [_S1_MATS[8]],
                                         [wn1, wn2, wn3, wltr, wi1r, wi2r, wlr],
                                         [64, 32, 1, 64, 32, 32, 32]):
            wT = wref[...].T
            di = wT.shape[0]
            for u in range(F):
                scr[pl.ds(off + di * u, di), do * u:do * u + do] = wT
        for off, wv in ((1800, whwr[:, 0:64]), (2056, whwr[:, 64:96])):
            wT = wv.T
            di = wT.shape[0]
            for u in range(F):
                for k in range(3):
                    scr[pl.ds(off + di * u, di),
                        128 * k + 32 * u:128 * k + 32 * u + 32] = wT[:, 32 * k:32 * k + 32]

    def mat(i):
        off, r, c = _S1_MATS[i]
        return scr[pl.ds(off, r), 0:c]

    def bias(i, c):
        return scr[pl.ds(i, 1), 0:c]

    a1, a2, a3 = mat(0), mat(1), mat(2)
    wlt, wi1, wi2, wih, whh, wl = mat(3), mat(4), mat(5), mat(6), mat(7), mat(8)
    c1, c2, c3 = bias(0, 256), bias(1, 128), bias(2, 4)
    bi1, bi2 = bias(3, 128), bias(4, 128)
    bih, bhh, bl = bias(5, 384), bias(6, 384), bias(7, 128)

    lat = lat_ref[...].reshape(lat_ref.shape[0] // F, F * lat_ref.shape[1])
    h = _lk(lat @ a1 + c1)
    h = _lk(h @ a2 + c2)
    nn_ref[...] = h @ a3 + c3

    proj = jax.nn.sigmoid(lat @ wlt)
    hs = jnp.maximum(proj @ wi1 + bi1, 0.0)
    hs = hs @ wi2 + bi2
    gi = proj @ wih + bih
    W = F * RNN_H
    ir = gi[:, 0:W]
    iz = gi[:, W:2 * W]
    inn = gi[:, 2 * W:3 * W]
    for t in range(MAX_NODES):
        gh = hs @ whh + bhh
        hr = gh[:, 0:W]
        hz = gh[:, W:2 * W]
        hn = gh[:, 2 * W:3 * W]
        r = jax.nn.sigmoid(ir + hr)
        z = jax.nn.sigmoid(iz + hz)
        n = jnp.tanh(inn + r * hn)
        hs = (1.0 - z) * n + z * hs
        st_ref[t] = hs @ wl + bl


def _run_stage1(latent, nn_mlp, Wlt, gru):
    Bn, L = latent.shape
    R = Bn // F
    BLK = 1024
    grid = (R // BLK,)
    Wn1, bn1, Wn2, bn2, Wn3, bn3 = nn_mlp
    Wi1, bi1, Wi2, bi2, Wih, Whh, bih, bhh, Wl, bl = gru
    raw = [Wn1, bn1[None, :], Wn2, bn2[None, :], Wn3, bn3[None, :], Wlt,
           Wi1, bi1[None, :], Wi2, bi2[None, :],
           jnp.concatenate([Wih, Whh], axis=1),
           bih[None, :], bhh[None, :], Wl, bl[None, :]]

    def full(a):
        return pl.BlockSpec(a.shape, lambda i: (0,) * a.ndim)

    nn4, states = pl.pallas_call(
        _stage1_body,
        grid=grid,
        in_specs=[pl.BlockSpec((F * BLK, L), lambda i: (i, 0))] + [full(a) for a in raw],
        out_specs=[pl.BlockSpec((BLK, F), lambda i: (i, 0)),
                   pl.BlockSpec((MAX_NODES, BLK, F * RNN_H), lambda i: (0, i, 0))],
        out_shape=[jax.ShapeDtypeStruct((R, F), jnp.float32),
                   jax.ShapeDtypeStruct((MAX_NODES, R, F * RNN_H), jnp.float32)],
        scratch_shapes=[pltpu.VMEM((_S1_ROWS, 384), jnp.float32)],
    )(latent, *raw)
    return nn4.reshape(-1), states


def _run_sc_window_sum(flat128, B, per):
    NW = 32
    gpw = B // NW
    grp = gpw // 4
    rows = grp * MAX_NODES
    mesh = plsc.VectorSubcoreMesh(core_axis_name="c", subcore_axis_name="s")

    NCH = 4
    CH = rows // NCH
    GC = grp // NCH

    @functools.partial(
        pl.kernel,
        out_type=jax.ShapeDtypeStruct((B, RNN_H), jnp.float32),
        mesh=mesh,
        scratch_types=[pltpu.VMEM((2, CH, 128), jnp.float32),
                       pltpu.VMEM((gpw, RNN_H), jnp.float32),
                       pltpu.SemaphoreType.DMA,
                       pltpu.SemaphoreType.DMA],
    )
    def _sc_sum(flat_hbm, out_hbm, bufs, acc_v, sem0, sem1):
        wid = lax.axis_index("s") * 2 + lax.axis_index("c")
        base = wid * rows
        sems = (sem0, sem1)
        handles = [None, None]
        handles[0] = pltpu.async_copy(flat_hbm.at[pl.ds(base, CH)], bufs.at[0], sems[0])
        for c in range(NCH):
            nb = (c + 1) % 2
            if c + 1 < NCH:
                handles[nb] = pltpu.async_copy(
                    flat_hbm.at[pl.ds(base + (c + 1) * CH, CH)], bufs.at[nb], sems[nb])
            handles[c % 2].wait()

            def body(q, carry, c=c):
                rbase = q * MAX_NODES
                for d in range(4):
                    for h in range(2):
                        p0 = 32 * MAX_NODES * d + 16 * h
                        acc = bufs[c % 2, rbase + p0 // 128, pl.ds(p0 % 128, 16)]
                        for j in range(1, per):
                            p = p0 + 32 * j
                            acc = acc + bufs[c % 2, rbase + p // 128, pl.ds(p % 128, 16)]
                        acc_v[4 * (GC * c + q) + d, pl.ds(16 * h, 16)] = acc
                return carry

            lax.fori_loop(0, GC, body, 0)
        pltpu.sync_copy(acc_v, out_hbm.at[pl.ds(wid * gpw, gpw)])

    return _sc_sum(flat128)


def _stage3_body(offs, per, *refs):
    s_ref = refs[0]
    colp = refs[1]
    m128 = refs[2]
    out_ref = refs[-2]
    ot_scr = refs[-1]
    wrefs = refs[3:-2]
    scale = float(per)

    @pl.when(pl.program_id(0) == 0)
    def _compute():
        def col(i):
            return colp[pl.ds(offs[i], offs[i + 1] - offs[i]), :]

        xT = s_ref[...].T
        k = 0
        c = 0
        for i in range(4):
            has_m = i < 3
            if i == 2:
                pair = wrefs[k][...]
                w1, w2 = pair[:, 0:32], pair[:, 32:96]
                k += 1
            else:
                w1 = wrefs[k][...]
                w2 = wrefs[k + 1][...]
                k += 2
            if i == 0:
                w1 = w1[:, 0:RNN_H]
                z = w1 @ xT
            else:
                z = w1 @ (scale * xT)
            b1, g1, bt1, b2 = col(c), col(c + 1), col(c + 2), col(c + 3)
            z = z + b1
            m = jnp.mean(z, axis=1, keepdims=True)
            zc = z - m
            v = jnp.mean(zc * zc, axis=1, keepdims=True)
            z = _lk(zc * (g1 / jnp.sqrt(v + 1e-5)) + bt1)
            z = w2 @ z + b2
            if has_m:
                mg, mb = col(c + 4), col(c + 5)
                c += 6
                m = jnp.mean(z, axis=1, keepdims=True)
                zc = z - m
                v = jnp.mean(zc * zc, axis=1, keepdims=True)
                z = _lk(zc * (mg / jnp.sqrt(v + 1e-5)) + mb)
            else:
                c += 4
            xT = z
        f1, f2, f3 = wrefs[k][...], wrefs[k + 1][...], wrefs[k + 2][...]
        o = _lk(f1 @ xT)
        o = _lk(f2 @ o)
        ot_scr[...] = f3 @ o

    qcols = out_ref.shape[1] // (128 * per)
    base = pl.program_id(0) * qcols
    for t in range(qcols):
        out_ref[:, per * 128 * t:per * 128 * (t + 1)] = (
            ot_scr[:, pl.ds(128 * (base + t), 128)] @ m128[...])


def _run_stage3(S, gin, fin, B, per):
    wmats, cols, offs = [], [], [0]
    for li, layer in enumerate(gin):
        if len(layer) == 8:
            W1, b1, g1, bt1, W2, b2, mg, mb = layer
            vecs = [b1, g1, bt1, b2, mg, mb]
        else:
            W1, b1, g1, bt1, W2, b2 = layer
            vecs = [b1, g1, bt1, b2]
        if li == 2:
            wmats += [jnp.concatenate([W1, W2], axis=1)]
        else:
            wmats += [W1, W2]
        for vct in vecs:
            cols.append(vct)
            offs.append(offs[-1] + vct.shape[0])
    Wf1, Wf2, Wf3 = fin
    wmats += [Wf1, Wf2, Wf3]
    colp = jnp.concatenate(cols)[:, None]

    odim = Wf3.shape[0]
    m128 = np.zeros((128, 128 * per), np.float32)
    m128[np.arange(128).repeat(per), np.arange(128 * per)] = 1.0
    m128 = jnp.asarray(m128)

    def full(a):
        return pl.BlockSpec(a.shape, lambda i: (0,) * a.ndim)

    NQ = 4
    out_t = pl.pallas_call(
        functools.partial(_stage3_body, tuple(offs), per),
        grid=(NQ,),
        in_specs=[full(S), full(colp), full(m128)] + [full(a) for a in wmats],
        out_specs=pl.BlockSpec((odim, B * per // NQ), lambda i: (0, i)),
        out_shape=jax.ShapeDtypeStruct((odim, B * per), jnp.float32),
        scratch_shapes=[pltpu.VMEM((odim, B), jnp.float32)],
    )(S, colp, m128, *wmats)
    return out_t.T


def _edge_index_np(B, per):
    ii, jj = np.meshgrid(np.arange(per), np.arange(per), indexing="ij")
    m = ii != jj
    offs = (np.arange(B, dtype=np.int64) * per)[:, None]
    src = (ii[m][None, :] + offs).reshape(-1)
    dst = (jj[m][None, :] + offs).reshape(-1)
    return np.stack([src, dst]).astype(np.int32)


def kernel(latent_vec, batch, nn_mlp, Wlt, gru, gin, fin):
    B, L = latent_vec.shape
    per = batch.shape[0] // B
    edge_index = jnp.asarray(_edge_index_np(B, per))

    number_nodes, states = _run_stage1(latent_vec, nn_mlp, Wlt, gru)
    flat128 = states.reshape(MAX_NODES * B // F, F * RNN_H)
    S = _run_sc_window_sum(flat128, B, per)
    out = _run_stage3(S, gin, fin, B, per)
    return out, edge_index, batch, number_nodes
